# baseline (speedup 1.0000x reference)
import functools

import jax
import jax.numpy as jnp
from jax import lax
from jax.experimental import pallas as pl
from jax.experimental.pallas import tpu as pltpu
from jax.experimental.pallas import tpu_sc as plsc

_CHUNK = 256
_NBUF = 2


@functools.cache
def _make_gather(n_rows, n_vocab, d):
    info = plsc.get_sparse_core_info()
    nc, ns = info.num_cores, info.num_subcores
    nw = nc * ns
    b_per_w = n_rows // nw
    n_chunks = b_per_w // _CHUNK
    n_groups = n_chunks // _NBUF
    mesh = plsc.VectorSubcoreMesh(core_axis_name="c", subcore_axis_name="s")

    @functools.partial(
        pl.kernel,
        mesh=mesh,
        compiler_params=pltpu.CompilerParams(needs_layout_passes=False),
        out_type=jax.ShapeDtypeStruct((n_rows, d), jnp.float32),
        scratch_types=[
            pltpu.VMEM_SHARED((n_vocab, d), jnp.float32),
            pltpu.VMEM((b_per_w,), jnp.int32),
            pltpu.VMEM((_NBUF, _CHUNK, d), jnp.float32),
            pltpu.SemaphoreType.DMA((_NBUF,)),
            pltpu.SemaphoreType.DMA((_NBUF,)),
        ],
    )
    def gather_kernel(
        tab_hbm, idx_hbm, out_hbm, table_sh, idx_v, rows_v, sem_g, sem_o
    ):
        sid = lax.axis_index("s")
        wid = sid * nc + lax.axis_index("c")
        base = wid * b_per_w

        @pl.when(sid == 0)
        def _():
            pltpu.sync_copy(tab_hbm, table_sh)

        pltpu.sync_copy(idx_hbm.at[pl.ds(base, b_per_w)], idx_v)
        plsc.subcore_barrier()

        def g_copy(i, b):
            return pltpu.make_async_copy(
                table_sh.at[idx_v.at[pl.ds(i * _CHUNK, _CHUNK)]],
                rows_v.at[b],
                sem_g.at[b],
            )

        def o_copy(i, b):
            return pltpu.make_async_copy(
                rows_v.at[b],
                out_hbm.at[pl.ds(base + i * _CHUNK, _CHUNK)],
                sem_o.at[b],
            )

        def step(i, b, wait_prev, start_next):
            g_copy(i, b).wait()
            o_copy(i, b).start()
            if wait_prev:
                o_copy(i - 1, (b - 1) % _NBUF).wait()
            if start_next:
                g_copy(i + _NBUF - 1, (b + _NBUF - 1) % _NBUF).start()

        for b in range(_NBUF - 1):
            g_copy(b, b).start()

        for b in range(_NBUF):
            step(b, b, wait_prev=(b > 0), start_next=True)

        def group(j, carry):
            i0 = j * _NBUF
            for b in range(_NBUF):
                step(i0 + b, b, wait_prev=True, start_next=True)
            return carry

        lax.fori_loop(1, n_groups - 1, group, 0)

        i0 = (n_groups - 1) * _NBUF
        step(i0, 0, wait_prev=True, start_next=True)
        for b in range(1, _NBUF):
            step(i0 + b, b, wait_prev=False, start_next=False)

        for b in range(_NBUF):
            o_copy(i0 + b, b).wait()

    return gather_kernel


def kernel(seq_indices, pe):
    batch, seq_len = seq_indices.shape
    d = pe.shape[-1]
    n_vocab = pe.shape[1]
    n_rows = batch * seq_len
    flat_idx = seq_indices.reshape(n_rows)
    table = pe[0]
    out = _make_gather(n_rows, n_vocab, d)(table, flat_idx)
    return out.reshape(batch, seq_len, d)


# speedup vs baseline: 1.0420x; 1.0420x over previous
import functools

import jax
import jax.numpy as jnp
from jax import lax
from jax.experimental import pallas as pl
from jax.experimental.pallas import tpu as pltpu
from jax.experimental.pallas import tpu_sc as plsc

_CHUNK = 128
_NBUF = 4


@functools.cache
def _make_gather(n_rows, n_vocab, d):
    info = plsc.get_sparse_core_info()
    nc, ns = info.num_cores, info.num_subcores
    nw = nc * ns
    b_per_w = n_rows // nw
    n_chunks = b_per_w // _CHUNK
    n_groups = n_chunks // _NBUF
    mesh = plsc.VectorSubcoreMesh(core_axis_name="c", subcore_axis_name="s")

    @functools.partial(
        pl.kernel,
        mesh=mesh,
        compiler_params=pltpu.CompilerParams(needs_layout_passes=False),
        out_type=jax.ShapeDtypeStruct((n_rows, d), jnp.float32),
        scratch_types=[
            pltpu.VMEM_SHARED((n_vocab, d), jnp.float32),
            pltpu.VMEM((b_per_w,), jnp.int32),
            pltpu.VMEM((_NBUF, _CHUNK, d), jnp.float32),
            pltpu.SemaphoreType.DMA((_NBUF,)),
            pltpu.SemaphoreType.DMA((_NBUF,)),
        ],
    )
    def gather_kernel(
        tab_hbm, idx_hbm, out_hbm, table_sh, idx_v, rows_v, sem_g, sem_o
    ):
        sid = lax.axis_index("s")
        wid = sid * nc + lax.axis_index("c")
        base = wid * b_per_w

        s_sz = (n_vocab + ns - 1) // ns
        s_last = n_vocab - (ns - 1) * s_sz
        idx_cp = pltpu.make_async_copy(
            idx_hbm.at[pl.ds(base, b_per_w)], idx_v, sem_o.at[0]
        )
        idx_cp.start()

        @pl.when(sid < ns - 1)
        def _():
            pltpu.sync_copy(
                tab_hbm.at[pl.ds(sid * s_sz, s_sz)],
                table_sh.at[pl.ds(sid * s_sz, s_sz)],
            )

        @pl.when(sid == ns - 1)
        def _():
            pltpu.sync_copy(
                tab_hbm.at[pl.ds((ns - 1) * s_sz, s_last)],
                table_sh.at[pl.ds((ns - 1) * s_sz, s_last)],
            )

        idx_cp.wait()
        plsc.subcore_barrier()

        def g_copy(i, b):
            return pltpu.make_async_copy(
                table_sh.at[idx_v.at[pl.ds(i * _CHUNK, _CHUNK)]],
                rows_v.at[b],
                sem_g.at[b],
            )

        def o_copy(i, b):
            return pltpu.make_async_copy(
                rows_v.at[b],
                out_hbm.at[pl.ds(base + i * _CHUNK, _CHUNK)],
                sem_o.at[b],
            )

        def step(i, b, wait_prev, start_next):
            g_copy(i, b).wait()
            o_copy(i, b).start()
            if wait_prev:
                o_copy(i - 1, (b - 1) % _NBUF).wait()
            if start_next:
                g_copy(i + _NBUF - 1, (b + _NBUF - 1) % _NBUF).start()

        for b in range(_NBUF - 1):
            g_copy(b, b).start()

        for b in range(_NBUF):
            step(b, b, wait_prev=(b > 0), start_next=True)

        def group(j, carry):
            i0 = j * _NBUF
            for b in range(_NBUF):
                step(i0 + b, b, wait_prev=True, start_next=True)
            return carry

        lax.fori_loop(1, n_groups - 1, group, 0)

        i0 = (n_groups - 1) * _NBUF
        step(i0, 0, wait_prev=True, start_next=True)
        for b in range(1, _NBUF):
            step(i0 + b, b, wait_prev=False, start_next=False)

        for b in range(_NBUF):
            o_copy(i0 + b, b).wait()

    return gather_kernel


def kernel(seq_indices, pe):
    batch, seq_len = seq_indices.shape
    d = pe.shape[-1]
    n_vocab = pe.shape[1]
    n_rows = batch * seq_len
    flat_idx = seq_indices.reshape(n_rows)
    table = pe[0]
    out = _make_gather(n_rows, n_vocab, d)(table, flat_idx)
    return out.reshape(batch, seq_len, d)
